# baseline (speedup 1.0000x reference)
import functools

import jax
import jax.numpy as jnp
from jax import lax
from jax.experimental import pallas as pl
from jax.experimental.pallas import tpu as pltpu
from jax.experimental.pallas import tpu_sc as plsc

N = 50000
NP = 50176
E = 800000
EP = 802816
R = EP // 128
RT = R // 16
PT = NP // 16
SCH = 56
EJ = 56
NHOPS = 10
TN = 256
NT = NP // TN

_mesh = plsc.VectorSubcoreMesh(core_axis_name="c", subcore_axis_name="s")
f32 = jnp.float32


def _degrees_body(cols4, ones_h, zeros_h, deg_out, cidx, cvec, ones, zbuf, acc):
    cid = lax.axis_index("c")
    sid = lax.axis_index("s")
    pltpu.sync_copy(ones_h, ones)
    pltpu.sync_copy(zeros_h, zbuf)
    for a in range(2):
        pltpu.sync_copy(cols4.at[cid, a, pl.ds(sid * RT, RT)], cidx)

        def _zero(m, carry):
            pltpu.sync_copy(zbuf, acc.at[pl.ds(sid * PT + m * SCH, SCH)])
            return carry

        lax.fori_loop(0, PT // SCH, _zero, 0)
        plsc.subcore_barrier()

        def _edges(j, carry):
            for t in range(8):
                cvec[pl.ds(t * 16, 16)] = cidx[j, pl.ds(t * 16, 16)]
            pltpu.sync_copy(ones, acc.at[cvec], add=True)
            return carry

        lax.fori_loop(0, RT, _edges, 0)
        plsc.subcore_barrier()
        pltpu.sync_copy(acc.at[pl.ds(sid * PT, PT)],
                        deg_out.at[cid, a, pl.ds(sid * PT, PT)])
        plsc.subcore_barrier()


def _degrees(cols4, ones_h, zeros_h):
    return pl.kernel(
        _degrees_body,
        out_type=jax.ShapeDtypeStruct((2, 2, NP), f32),
        mesh=_mesh,
        compiler_params=pltpu.CompilerParams(use_tc_tiling_on_sc=False),
        scratch_types=[
            pltpu.VMEM((RT, 128), jnp.int32),
            pltpu.VMEM((128,), jnp.int32),
            pltpu.VMEM((128,), f32),
            pltpu.VMEM((SCH,), f32),
            pltpu.VMEM_SHARED((NP,), f32),
        ],
    )(cols4, ones_h, zeros_h)


def _edge_pass(rows3, cols3, blk, srcb, acc, ridx, cidx, buf0, buf1,
               sem0, sem1, sid):
    def _echunk(c, c2):
        pltpu.sync_copy(rows3.at[blk, pl.ds(sid * RT + c * EJ, EJ)], ridx)
        pltpu.sync_copy(cols3.at[blk, pl.ds(sid * RT + c * EJ, EJ)], cidx)

        def _pair(i, c3):
            j0 = 2 * i
            j1 = j0 + 1
            g0 = pltpu.async_copy(srcb.at[ridx.at[j0]], buf0, sem0)
            g1 = pltpu.async_copy(srcb.at[ridx.at[j1]], buf1, sem1)
            g0.wait()
            pltpu.sync_copy(buf0, acc.at[cidx.at[j0]], add=True)
            g1.wait()
            pltpu.sync_copy(buf1, acc.at[cidx.at[j1]], add=True)
            return c3

        lax.fori_loop(0, EJ // 2, _pair, 0)
        return c2

    lax.fori_loop(0, RT // EJ, _echunk, 0)


def _propagate_body(NB, rows3, cols3, q0, drep, zeros_h, q_out,
                    ridx, cidx, buf0, buf1, zbuf, acch, dreph, srcb, acc,
                    sem0, sem1):
    cid = lax.axis_index("c")
    sid = lax.axis_index("s")
    pltpu.sync_copy(zeros_h, zbuf)
    myslice = pl.ds(sid * PT, PT)

    def _sub_pass(k, f):
        blk = cid * NB + f
        if k is None:
            pltpu.sync_copy(q0.at[blk, myslice], srcb.at[myslice])
        else:
            pltpu.sync_copy(q_out.at[k - 2, blk, myslice], srcb.at[myslice])

        def _zero(m, c2):
            pltpu.sync_copy(zbuf, acc.at[pl.ds(sid * PT + m * SCH, SCH)])
            return c2

        lax.fori_loop(0, PT // SCH, _zero, 0)
        plsc.subcore_barrier()

        _edge_pass(rows3, cols3, blk, srcb, acc, ridx, cidx, buf0, buf1,
                   sem0, sem1, sid)
        plsc.subcore_barrier()

        kk = 1 if k is None else k

        def _chunk(m, c2):
            r0 = sid * PT + m * SCH
            pltpu.sync_copy(acc.at[pl.ds(r0, SCH)], acch)
            pltpu.sync_copy(drep.at[blk, pl.ds(r0, SCH)], dreph)

            def _scale(r, c3):
                acch[r] = acch[r] * dreph[r]
                return c3

            lax.fori_loop(0, SCH, _scale, 0)
            pltpu.sync_copy(acch, q_out.at[kk - 1, blk, pl.ds(r0, SCH)])
            return c2

        lax.fori_loop(0, PT // SCH, _chunk, 0)
        plsc.subcore_barrier()

    for f in range(NB):
        _sub_pass(None, f)

    def _hop(k, carry):
        for f in range(NB):
            _sub_pass(k, f)
        return carry

    lax.fori_loop(2, NHOPS + 1, _hop, 0)


def _propagate(NB, rows3, cols3, q0, drep, zeros_h):
    return pl.kernel(
        functools.partial(_propagate_body, NB),
        out_type=jax.ShapeDtypeStruct((NHOPS, 2 * NB, NP, 16), f32),
        mesh=_mesh,
        compiler_params=pltpu.CompilerParams(use_tc_tiling_on_sc=False),
        scratch_types=[
            pltpu.VMEM((EJ, 128), jnp.int32),
            pltpu.VMEM((EJ, 128), jnp.int32),
            pltpu.VMEM((128, 16), f32),
            pltpu.VMEM((128, 16), f32),
            pltpu.VMEM((SCH, 16), f32),
            pltpu.VMEM((SCH, 16), f32),
            pltpu.VMEM((SCH, 16), f32),
            pltpu.VMEM_SHARED((NP, 16), f32),
            pltpu.VMEM_SHARED((NP, 16), f32),
            pltpu.SemaphoreType.DMA,
            pltpu.SemaphoreType.DMA,
        ],
    )(rows3, cols3, q0, drep, zeros_h)


def _sage_body(rows_h, cols_h, rows_i, cols_i, g2, zeros_h, agg_out,
               ridx, cidx, buf0, buf1, zbuf, srcb, acc, sem0, sem1):
    cid = lax.axis_index("c")
    sid = lax.axis_index("s")
    pltpu.sync_copy(zeros_h, zbuf)
    myslice = pl.ds(sid * PT, PT)
    for a in range(2):
        rows3 = rows_h if a == 0 else rows_i
        cols3 = cols_h if a == 0 else cols_i
        for f in range(2):
            blk = cid * 2 + f
            pltpu.sync_copy(g2.at[blk, myslice], srcb.at[myslice])

            def _zero(m, c2):
                pltpu.sync_copy(zbuf, acc.at[pl.ds(sid * PT + m * SCH, SCH)])
                return c2

            lax.fori_loop(0, PT // SCH, _zero, 0)
            plsc.subcore_barrier()
            _edge_pass(rows3, cols3, blk, srcb, acc, ridx, cidx, buf0, buf1,
                       sem0, sem1, sid)
            plsc.subcore_barrier()
            pltpu.sync_copy(acc.at[myslice], agg_out.at[a, blk, myslice])
            plsc.subcore_barrier()


def _sage_agg(rows_h, cols_h, rows_i, cols_i, g2, zeros_h):
    return pl.kernel(
        _sage_body,
        out_type=jax.ShapeDtypeStruct((2, 4, NP, 16), f32),
        mesh=_mesh,
        compiler_params=pltpu.CompilerParams(use_tc_tiling_on_sc=False),
        scratch_types=[
            pltpu.VMEM((EJ, 128), jnp.int32),
            pltpu.VMEM((EJ, 128), jnp.int32),
            pltpu.VMEM((128, 16), f32),
            pltpu.VMEM((128, 16), f32),
            pltpu.VMEM((SCH, 16), f32),
            pltpu.VMEM_SHARED((NP, 16), f32),
            pltpu.VMEM_SHARED((NP, 16), f32),
            pltpu.SemaphoreType.DMA,
            pltpu.SemaphoreType.DMA,
        ],
    )(rows_h, cols_h, rows_i, cols_i, g2, zeros_h)


def _prep_body(deg_ref, xg_ref, xs_ref, scl_ref, drep_ref, q0_ref):
    dgg = deg_ref[0]
    dh = deg_ref[1]
    dss = deg_ref[2]
    din = deg_ref[3]
    rs_gg = jnp.where(dgg > 0, lax.rsqrt(dgg), 0.0)
    rs_ss = jnp.where(dss > 0, lax.rsqrt(dss), 0.0)
    scl_ref[0] = jnp.where(dgg > 0, jnp.sqrt(dgg), 0.0)
    scl_ref[1] = jnp.where(dss > 0, jnp.sqrt(dss), 0.0)
    scl_ref[2] = 1.0 / jnp.maximum(dh, 1.0)
    scl_ref[3] = 1.0 / jnp.maximum(din, 1.0)
    scl_ref[4] = rs_gg
    scl_ref[5] = rs_ss
    drep_ref[0] = jnp.broadcast_to(jnp.where(dgg > 0, 1.0 / dgg, 0.0), (TN, 16))
    drep_ref[1] = jnp.broadcast_to(jnp.where(dss > 0, 1.0 / dss, 0.0), (TN, 16))
    q0_ref[0] = xg_ref[...] * rs_gg
    q0_ref[1] = xs_ref[...] * rs_ss


def _prep(deg4, xg, xs):
    return pl.pallas_call(
        _prep_body,
        grid=(NT,),
        in_specs=[
            pl.BlockSpec((4, TN, 1), lambda i: (0, i, 0)),
            pl.BlockSpec((TN, 16), lambda i: (i, 0)),
            pl.BlockSpec((TN, 16), lambda i: (i, 0)),
        ],
        out_specs=[
            pl.BlockSpec((6, TN, 1), lambda i: (0, i, 0)),
            pl.BlockSpec((2, TN, 16), lambda i: (0, i, 0)),
            pl.BlockSpec((2, TN, 16), lambda i: (0, i, 0)),
        ],
        out_shape=[
            jax.ShapeDtypeStruct((6, NP, 1), f32),
            jax.ShapeDtypeStruct((2, NP, 16), f32),
            jax.ShapeDtypeStruct((2, NP, 16), f32),
        ],
    )(deg4, xg, xs)


def _dot(a, b):
    return jnp.dot(a, b, preferred_element_type=f32)


def _mmn_body(xg_ref, xs_ref, qn_ref, scl_ref, w1_ref, b1_ref, w2_ref, b2_ref,
              gx_ref, gq0_ref, sx_ref, sq0_ref):
    for (x, w, b, d12, dinv, ox, oq, c) in (
            (xg_ref, w1_ref, b1_ref, scl_ref[0], scl_ref[4], gx_ref, gq0_ref, 0),
            (xs_ref, w2_ref, b2_ref, scl_ref[1], scl_ref[5], sx_ref, sq0_ref, 1)):
        acc0 = _dot(x[...], w[0])
        accp = _dot(qn_ref[0, c], w[1])
        for k in range(2, NHOPS + 1):
            accp = accp + _dot(qn_ref[k - 1, c], w[k])
        o = jax.nn.relu(acc0 + d12 * accp + b[...])
        oq2 = o * dinv
        for blk in range(4):
            ox[blk] = o[:, blk * 16:(blk + 1) * 16]
            oq[blk] = oq2[:, blk * 16:(blk + 1) * 16]


def _mmn(xg, xs, qn, scl, w1, b1, w2, b2):
    return pl.pallas_call(
        _mmn_body,
        grid=(NT,),
        in_specs=[
            pl.BlockSpec((TN, 16), lambda i: (i, 0)),
            pl.BlockSpec((TN, 16), lambda i: (i, 0)),
            pl.BlockSpec((NHOPS, 2, TN, 16), lambda i: (0, 0, i, 0)),
            pl.BlockSpec((6, TN, 1), lambda i: (0, i, 0)),
            pl.BlockSpec((NHOPS + 1, 16, 64), lambda i: (0, 0, 0)),
            pl.BlockSpec((1, 64), lambda i: (0, 0)),
            pl.BlockSpec((NHOPS + 1, 16, 64), lambda i: (0, 0, 0)),
            pl.BlockSpec((1, 64), lambda i: (0, 0)),
        ],
        out_specs=[pl.BlockSpec((4, TN, 16), lambda i: (0, i, 0))] * 4,
        out_shape=[jax.ShapeDtypeStruct((4, NP, 16), f32)] * 4,
    )(xg, xs, qn, scl, w1, b1, w2, b2)


def _mmw_body(srow, x_ref, q_ref, scl_ref, w_ref, b_ref, out_ref):
    acc0 = _dot(x_ref[0], w_ref[0, 0:16, :])
    for blk in range(1, 4):
        acc0 = acc0 + _dot(x_ref[blk], w_ref[0, blk * 16:(blk + 1) * 16, :])
    accp = None
    for k in range(1, NHOPS + 1):
        for blk in range(4):
            t = _dot(q_ref[k - 1, blk], w_ref[k, blk * 16:(blk + 1) * 16, :])
            accp = t if accp is None else accp + t
    o = jax.nn.relu(acc0 + scl_ref[srow] * accp + b_ref[...])
    for blk in range(4):
        out_ref[blk] = o[:, blk * 16:(blk + 1) * 16]


def _mmw(srow, x, q, scl, w, b):
    return pl.pallas_call(
        functools.partial(_mmw_body, srow),
        grid=(NT,),
        in_specs=[
            pl.BlockSpec((4, TN, 16), lambda i: (0, i, 0)),
            pl.BlockSpec((NHOPS, 4, TN, 16), lambda i: (0, 0, i, 0)),
            pl.BlockSpec((6, TN, 1), lambda i: (0, i, 0)),
            pl.BlockSpec((NHOPS + 1, 64, 64), lambda i: (0, 0, 0)),
            pl.BlockSpec((1, 64), lambda i: (0, 0)),
        ],
        out_specs=pl.BlockSpec((4, TN, 16), lambda i: (0, i, 0)),
        out_shape=jax.ShapeDtypeStruct((4, NP, 16), f32),
    )(x, q, scl, w, b)


def _mms_body(s2_ref, agg_ref, scl_ref,
              wl3, bl3, wr3, wl32, bl32, wr32,
              wl4, bl4, wr4, wl42, bl42, wr42, lw, lb, out_ref):
    s = jnp.concatenate([s2_ref[b] for b in range(4)], axis=1)
    mh = jnp.concatenate([agg_ref[0, b] for b in range(4)], axis=1) * scl_ref[2]
    mi = jnp.concatenate([agg_ref[1, b] for b in range(4)], axis=1) * scl_ref[3]
    s = jax.nn.relu(_dot(mh, wl3[...]) + bl3[...] + _dot(s, wr3[...]))
    s = jax.nn.relu(_dot(mh, wl32[...]) + bl32[...] + _dot(s, wr32[...]))
    s = jax.nn.relu(_dot(mi, wl4[...]) + bl4[...] + _dot(s, wr4[...]))
    s = jax.nn.relu(_dot(mi, wl42[...]) + bl42[...] + _dot(s, wr42[...]))
    out_ref[...] = _dot(s, lw[...]) + lb[...]


def _mms(s2, agg, scl, *ws):
    wspecs = []
    for w in ws:
        wspecs.append(pl.BlockSpec(w.shape, lambda i, nd=w.ndim: (0,) * nd))
    return pl.pallas_call(
        _mms_body,
        grid=(NT,),
        in_specs=[
            pl.BlockSpec((4, TN, 16), lambda i: (0, i, 0)),
            pl.BlockSpec((2, 4, TN, 16), lambda i: (0, 0, i, 0)),
            pl.BlockSpec((6, TN, 1), lambda i: (0, i, 0)),
        ] + wspecs,
        out_specs=pl.BlockSpec((TN, 8), lambda i: (i, 0)),
        out_shape=jax.ShapeDtypeStruct((NP, 8), f32),
    )(s2, agg, scl, *ws)


def _pad_edges(ei):
    r = jnp.concatenate([ei[0], jnp.zeros((EP - E,), jnp.int32)])
    c = jnp.concatenate([ei[1], jnp.full((EP - E,), N, jnp.int32)])
    return r.reshape(R, 128), c.reshape(R, 128)


def kernel(x_game, x_state, edge_gg, edge_ss, edge_hist, edge_in,
           tag1_W, tag1_b, tag12_W, tag12_b, tag2_W, tag2_b, tag22_W, tag22_b,
           s3_Wl, s3_bl, s3_Wr, s32_Wl, s32_bl, s32_Wr,
           s4_Wl, s4_bl, s4_Wr, s42_Wl, s42_bl, s42_Wr, lin_W, lin_b):
    xg = jnp.zeros((NP, 16), f32).at[:N, :5].set(x_game)
    xs = jnp.zeros((NP, 16), f32).at[:N, :6].set(x_state)
    w1 = jnp.zeros((NHOPS + 1, 16, 64), f32).at[:, :5, :].set(tag1_W)
    w2 = jnp.zeros((NHOPS + 1, 16, 64), f32).at[:, :6, :].set(tag2_W)

    gg_r, gg_c = _pad_edges(edge_gg)
    ss_r, ss_c = _pad_edges(edge_ss)
    h_r, h_c = _pad_edges(edge_hist)
    i_r, i_c = _pad_edges(edge_in)

    ones1 = jnp.ones((128,), f32)
    zeros1 = jnp.zeros((SCH,), f32)
    zeros16 = jnp.zeros((SCH, 16), f32)

    cols4 = jnp.stack([jnp.stack([gg_c, h_c]), jnp.stack([ss_c, i_c])])
    deg4 = _degrees(cols4, ones1, zeros1).reshape(4, NP, 1)

    scl, drep2, q0n = _prep(deg4, xg, xs)

    rows_n = jnp.stack([gg_r, ss_r])
    cols_n = jnp.stack([gg_c, ss_c])
    qn = _propagate(1, rows_n, cols_n, q0n, drep2, zeros16)

    gx, gq0, sx, sq0 = _mmn(xg, xs, qn, scl, w1, tag1_b.reshape(1, 64),
                            w2, tag2_b.reshape(1, 64))

    drep_gg = jnp.stack([drep2[0]] * 4)
    drep_ss = jnp.stack([drep2[1]] * 4)
    gg_r4 = jnp.stack([gg_r] * 4)
    gg_c4 = jnp.stack([gg_c] * 4)
    ss_r4 = jnp.stack([ss_r] * 4)
    ss_c4 = jnp.stack([ss_c] * 4)

    qg = _propagate(2, gg_r4, gg_c4, gq0, drep_gg, zeros16)
    g2 = _mmw(0, gx, qg, scl, tag12_W, tag12_b.reshape(1, 64))

    qs = _propagate(2, ss_r4, ss_c4, sq0, drep_ss, zeros16)
    s2 = _mmw(1, sx, qs, scl, tag22_W, tag22_b.reshape(1, 64))

    h_r4 = jnp.stack([h_r] * 4)
    h_c4 = jnp.stack([h_c] * 4)
    i_r4 = jnp.stack([i_r] * 4)
    i_c4 = jnp.stack([i_c] * 4)
    agg = _sage_agg(h_r4, h_c4, i_r4, i_c4, g2, zeros16)

    out = _mms(s2, agg, scl,
               s3_Wl, s3_bl.reshape(1, 64), s3_Wr,
               s32_Wl, s32_bl.reshape(1, 64), s32_Wr,
               s4_Wl, s4_bl.reshape(1, 64), s4_Wr,
               s42_Wl, s42_bl.reshape(1, 64), s42_Wr,
               lin_W, lin_b.reshape(1, 8))
    return out[:N]


# speedup vs baseline: 1.0795x; 1.0795x over previous
import functools

import jax
import jax.numpy as jnp
from jax import lax
from jax.experimental import pallas as pl
from jax.experimental.pallas import tpu as pltpu
from jax.experimental.pallas import tpu_sc as plsc

N = 50000
NP = 50176
E = 800000
EP = 802816
R = EP // 128
RT = R // 16
PT = NP // 16
SCH = 56
EJ = 56
NHOPS = 10
TN = 256
NT = NP // TN

_mesh = plsc.VectorSubcoreMesh(core_axis_name="c", subcore_axis_name="s")
f32 = jnp.float32


def _degrees_body(cols4, ones_h, zeros_h, deg_out, cidx, cvec, ones, zbuf, acc):
    cid = lax.axis_index("c")
    sid = lax.axis_index("s")
    pltpu.sync_copy(ones_h, ones)
    pltpu.sync_copy(zeros_h, zbuf)
    for a in range(2):
        pltpu.sync_copy(cols4.at[cid, a, pl.ds(sid * RT, RT)], cidx)

        def _zero(m, carry):
            pltpu.sync_copy(zbuf, acc.at[pl.ds(sid * PT + m * SCH, SCH)])
            return carry

        lax.fori_loop(0, PT // SCH, _zero, 0)
        plsc.subcore_barrier()

        def _edges(j, carry):
            for t in range(8):
                cvec[pl.ds(t * 16, 16)] = cidx[j, pl.ds(t * 16, 16)]
            pltpu.sync_copy(ones, acc.at[cvec], add=True)
            return carry

        lax.fori_loop(0, RT, _edges, 0)
        plsc.subcore_barrier()
        pltpu.sync_copy(acc.at[pl.ds(sid * PT, PT)],
                        deg_out.at[cid, a, pl.ds(sid * PT, PT)])
        plsc.subcore_barrier()


def _degrees(cols4, ones_h, zeros_h):
    return pl.kernel(
        _degrees_body,
        out_type=jax.ShapeDtypeStruct((2, 2, NP), f32),
        mesh=_mesh,
        compiler_params=pltpu.CompilerParams(use_tc_tiling_on_sc=False),
        scratch_types=[
            pltpu.VMEM((RT, 128), jnp.int32),
            pltpu.VMEM((128,), jnp.int32),
            pltpu.VMEM((128,), f32),
            pltpu.VMEM((SCH,), f32),
            pltpu.VMEM_SHARED((NP,), f32),
        ],
    )(cols4, ones_h, zeros_h)


def _edge_pass(rows3, cols3, blk, src, acc, ridx, cidx, bufs, gsems, ssems,
               sid):
    def _echunk(c, c2):
        pltpu.sync_copy(rows3.at[blk, pl.ds(sid * RT + c * EJ, EJ)], ridx)
        pltpu.sync_copy(cols3.at[blk, pl.ds(sid * RT + c * EJ, EJ)], cidx)
        gd = [None] * 8
        sd = [None] * 8
        for j in range(8):
            gd[j] = pltpu.async_copy(src.at[ridx.at[j]], bufs[j], gsems[j])
        for j in range(EJ):
            b = j % 8
            gd[b].wait()
            sd[b] = pltpu.async_copy(bufs[b], acc.at[cidx.at[j]], ssems[b],
                                     add=True)
            if 4 <= j < EJ - 4:
                bn = (j + 4) % 8
                sd[bn].wait()
                gd[bn] = pltpu.async_copy(src.at[ridx.at[j + 4]], bufs[bn],
                                          gsems[bn])
        for b in range(8):
            sd[b].wait()
        return c2

    lax.fori_loop(0, RT // EJ, _echunk, 0)


def _propagate_body(NB, rows3, cols3, q0, drep, zeros_h, q_out,
                    ridx, cidx, b0, b1, b2, b3, b4, b5, b6, b7,
                    zbuf, acch, dreph, acc,
                    g0, g1, g2, g3, g4, g5, g6, g7,
                    s0, s1, s2, s3, s4, s5, s6, s7):
    cid = lax.axis_index("c")
    sid = lax.axis_index("s")
    bufs = [b0, b1, b2, b3, b4, b5, b6, b7]
    gsems = [g0, g1, g2, g3, g4, g5, g6, g7]
    ssems = [s0, s1, s2, s3, s4, s5, s6, s7]
    pltpu.sync_copy(zeros_h, zbuf)

    def _sub_pass(k, f):
        blk = cid * NB + f
        src = q0.at[blk] if k is None else q_out.at[k - 2, blk]

        def _zero(m, c2):
            pltpu.sync_copy(zbuf, acc.at[pl.ds(sid * PT + m * SCH, SCH)])
            return c2

        lax.fori_loop(0, PT // SCH, _zero, 0)
        plsc.subcore_barrier()

        _edge_pass(rows3, cols3, blk, src, acc, ridx, cidx, bufs,
                   gsems, ssems, sid)
        plsc.subcore_barrier()

        kk = 1 if k is None else k

        def _chunk(m, c2):
            r0 = sid * PT + m * SCH
            pltpu.sync_copy(acc.at[pl.ds(r0, SCH)], acch)
            pltpu.sync_copy(drep.at[blk, pl.ds(r0, SCH)], dreph)

            def _scale(r, c3):
                acch[r] = acch[r] * dreph[r]
                return c3

            lax.fori_loop(0, SCH, _scale, 0)
            pltpu.sync_copy(acch, q_out.at[kk - 1, blk, pl.ds(r0, SCH)])
            return c2

        lax.fori_loop(0, PT // SCH, _chunk, 0)
        plsc.subcore_barrier()

    for f in range(NB):
        _sub_pass(None, f)

    def _hop(k, carry):
        for f in range(NB):
            _sub_pass(k, f)
        return carry

    lax.fori_loop(2, NHOPS + 1, _hop, 0)


def _propagate(NB, rows3, cols3, q0, drep, zeros_h):
    return pl.kernel(
        functools.partial(_propagate_body, NB),
        out_type=jax.ShapeDtypeStruct((NHOPS, 2 * NB, NP, 16), f32),
        mesh=_mesh,
        compiler_params=pltpu.CompilerParams(use_tc_tiling_on_sc=False),
        scratch_types=(
            [pltpu.VMEM((EJ, 128), jnp.int32)] * 2
            + [pltpu.VMEM((128, 16), f32)] * 8
            + [pltpu.VMEM((SCH, 16), f32)] * 3
            + [pltpu.VMEM_SHARED((NP, 16), f32)]
            + [pltpu.SemaphoreType.DMA] * 16
        ),
    )(rows3, cols3, q0, drep, zeros_h)


def _sage_body(rows_h, cols_h, rows_i, cols_i, gsrc, zeros_h, agg_out,
               ridx, cidx, b0, b1, b2, b3, b4, b5, b6, b7, zbuf, acc,
               g0, g1, g2, g3, g4, g5, g6, g7,
               s0, s1, s2, s3, s4, s5, s6, s7):
    cid = lax.axis_index("c")
    sid = lax.axis_index("s")
    bufs = [b0, b1, b2, b3, b4, b5, b6, b7]
    gsems = [g0, g1, g2, g3, g4, g5, g6, g7]
    ssems = [s0, s1, s2, s3, s4, s5, s6, s7]
    pltpu.sync_copy(zeros_h, zbuf)
    myslice = pl.ds(sid * PT, PT)
    for a in range(2):
        rows3 = rows_h if a == 0 else rows_i
        cols3 = cols_h if a == 0 else cols_i
        for f in range(2):
            blk = cid * 2 + f

            def _zero(m, c2):
                pltpu.sync_copy(zbuf, acc.at[pl.ds(sid * PT + m * SCH, SCH)])
                return c2

            lax.fori_loop(0, PT // SCH, _zero, 0)
            plsc.subcore_barrier()
            _edge_pass(rows3, cols3, blk, gsrc.at[blk], acc, ridx, cidx,
                       bufs, gsems, ssems, sid)
            plsc.subcore_barrier()
            pltpu.sync_copy(acc.at[myslice], agg_out.at[a, blk, myslice])
            plsc.subcore_barrier()


def _sage_agg(rows_h, cols_h, rows_i, cols_i, g2, zeros_h):
    return pl.kernel(
        _sage_body,
        out_type=jax.ShapeDtypeStruct((2, 4, NP, 16), f32),
        mesh=_mesh,
        compiler_params=pltpu.CompilerParams(use_tc_tiling_on_sc=False),
        scratch_types=(
            [pltpu.VMEM((EJ, 128), jnp.int32)] * 2
            + [pltpu.VMEM((128, 16), f32)] * 8
            + [pltpu.VMEM((SCH, 16), f32)] * 1
            + [pltpu.VMEM_SHARED((NP, 16), f32)]
            + [pltpu.SemaphoreType.DMA] * 16
        ),
    )(rows_h, cols_h, rows_i, cols_i, g2, zeros_h)


def _prep_body(deg_ref, xg_ref, xs_ref, scl_ref, drep_ref, q0_ref):
    dgg = deg_ref[0]
    dh = deg_ref[1]
    dss = deg_ref[2]
    din = deg_ref[3]
    rs_gg = jnp.where(dgg > 0, lax.rsqrt(dgg), 0.0)
    rs_ss = jnp.where(dss > 0, lax.rsqrt(dss), 0.0)
    scl_ref[0] = jnp.where(dgg > 0, jnp.sqrt(dgg), 0.0)
    scl_ref[1] = jnp.where(dss > 0, jnp.sqrt(dss), 0.0)
    scl_ref[2] = 1.0 / jnp.maximum(dh, 1.0)
    scl_ref[3] = 1.0 / jnp.maximum(din, 1.0)
    scl_ref[4] = rs_gg
    scl_ref[5] = rs_ss
    drep_ref[0] = jnp.broadcast_to(jnp.where(dgg > 0, 1.0 / dgg, 0.0), (TN, 16))
    drep_ref[1] = jnp.broadcast_to(jnp.where(dss > 0, 1.0 / dss, 0.0), (TN, 16))
    q0_ref[0] = xg_ref[...] * rs_gg
    q0_ref[1] = xs_ref[...] * rs_ss


def _prep(deg4, xg, xs):
    return pl.pallas_call(
        _prep_body,
        grid=(NT,),
        in_specs=[
            pl.BlockSpec((4, TN, 1), lambda i: (0, i, 0)),
            pl.BlockSpec((TN, 16), lambda i: (i, 0)),
            pl.BlockSpec((TN, 16), lambda i: (i, 0)),
        ],
        out_specs=[
            pl.BlockSpec((6, TN, 1), lambda i: (0, i, 0)),
            pl.BlockSpec((2, TN, 16), lambda i: (0, i, 0)),
            pl.BlockSpec((2, TN, 16), lambda i: (0, i, 0)),
        ],
        out_shape=[
            jax.ShapeDtypeStruct((6, NP, 1), f32),
            jax.ShapeDtypeStruct((2, NP, 16), f32),
            jax.ShapeDtypeStruct((2, NP, 16), f32),
        ],
    )(deg4, xg, xs)


def _dot(a, b):
    return jnp.dot(a, b, preferred_element_type=f32)


def _mmn_body(xg_ref, xs_ref, qn_ref, scl_ref, w1_ref, b1_ref, w2_ref, b2_ref,
              gx_ref, gq0_ref, sx_ref, sq0_ref):
    for (x, w, b, d12, dinv, ox, oq, c) in (
            (xg_ref, w1_ref, b1_ref, scl_ref[0], scl_ref[4], gx_ref, gq0_ref, 0),
            (xs_ref, w2_ref, b2_ref, scl_ref[1], scl_ref[5], sx_ref, sq0_ref, 1)):
        acc0 = _dot(x[...], w[0])
        accp = _dot(qn_ref[0, c], w[1])
        for k in range(2, NHOPS + 1):
            accp = accp + _dot(qn_ref[k - 1, c], w[k])
        o = jax.nn.relu(acc0 + d12 * accp + b[...])
        oq2 = o * dinv
        for blk in range(4):
            ox[blk] = o[:, blk * 16:(blk + 1) * 16]
            oq[blk] = oq2[:, blk * 16:(blk + 1) * 16]


def _mmn(xg, xs, qn, scl, w1, b1, w2, b2):
    return pl.pallas_call(
        _mmn_body,
        grid=(NT,),
        in_specs=[
            pl.BlockSpec((TN, 16), lambda i: (i, 0)),
            pl.BlockSpec((TN, 16), lambda i: (i, 0)),
            pl.BlockSpec((NHOPS, 2, TN, 16), lambda i: (0, 0, i, 0)),
            pl.BlockSpec((6, TN, 1), lambda i: (0, i, 0)),
            pl.BlockSpec((NHOPS + 1, 16, 64), lambda i: (0, 0, 0)),
            pl.BlockSpec((1, 64), lambda i: (0, 0)),
            pl.BlockSpec((NHOPS + 1, 16, 64), lambda i: (0, 0, 0)),
            pl.BlockSpec((1, 64), lambda i: (0, 0)),
        ],
        out_specs=[pl.BlockSpec((4, TN, 16), lambda i: (0, i, 0))] * 4,
        out_shape=[jax.ShapeDtypeStruct((4, NP, 16), f32)] * 4,
    )(xg, xs, qn, scl, w1, b1, w2, b2)


def _mmw_body(srow, x_ref, q_ref, scl_ref, w_ref, b_ref, out_ref):
    acc0 = _dot(x_ref[0], w_ref[0, 0:16, :])
    for blk in range(1, 4):
        acc0 = acc0 + _dot(x_ref[blk], w_ref[0, blk * 16:(blk + 1) * 16, :])
    accp = None
    for k in range(1, NHOPS + 1):
        for blk in range(4):
            t = _dot(q_ref[k - 1, blk], w_ref[k, blk * 16:(blk + 1) * 16, :])
            accp = t if accp is None else accp + t
    o = jax.nn.relu(acc0 + scl_ref[srow] * accp + b_ref[...])
    for blk in range(4):
        out_ref[blk] = o[:, blk * 16:(blk + 1) * 16]


def _mmw(srow, x, q, scl, w, b):
    return pl.pallas_call(
        functools.partial(_mmw_body, srow),
        grid=(NT,),
        in_specs=[
            pl.BlockSpec((4, TN, 16), lambda i: (0, i, 0)),
            pl.BlockSpec((NHOPS, 4, TN, 16), lambda i: (0, 0, i, 0)),
            pl.BlockSpec((6, TN, 1), lambda i: (0, i, 0)),
            pl.BlockSpec((NHOPS + 1, 64, 64), lambda i: (0, 0, 0)),
            pl.BlockSpec((1, 64), lambda i: (0, 0)),
        ],
        out_specs=pl.BlockSpec((4, TN, 16), lambda i: (0, i, 0)),
        out_shape=jax.ShapeDtypeStruct((4, NP, 16), f32),
    )(x, q, scl, w, b)


def _mms_body(s2_ref, agg_ref, scl_ref,
              wl3, bl3, wr3, wl32, bl32, wr32,
              wl4, bl4, wr4, wl42, bl42, wr42, lw, lb, out_ref):
    s = jnp.concatenate([s2_ref[b] for b in range(4)], axis=1)
    mh = jnp.concatenate([agg_ref[0, b] for b in range(4)], axis=1) * scl_ref[2]
    mi = jnp.concatenate([agg_ref[1, b] for b in range(4)], axis=1) * scl_ref[3]
    s = jax.nn.relu(_dot(mh, wl3[...]) + bl3[...] + _dot(s, wr3[...]))
    s = jax.nn.relu(_dot(mh, wl32[...]) + bl32[...] + _dot(s, wr32[...]))
    s = jax.nn.relu(_dot(mi, wl4[...]) + bl4[...] + _dot(s, wr4[...]))
    s = jax.nn.relu(_dot(mi, wl42[...]) + bl42[...] + _dot(s, wr42[...]))
    out_ref[...] = _dot(s, lw[...]) + lb[...]


def _mms(s2, agg, scl, *ws):
    wspecs = []
    for w in ws:
        wspecs.append(pl.BlockSpec(w.shape, lambda i, nd=w.ndim: (0,) * nd))
    return pl.pallas_call(
        _mms_body,
        grid=(NT,),
        in_specs=[
            pl.BlockSpec((4, TN, 16), lambda i: (0, i, 0)),
            pl.BlockSpec((2, 4, TN, 16), lambda i: (0, 0, i, 0)),
            pl.BlockSpec((6, TN, 1), lambda i: (0, i, 0)),
        ] + wspecs,
        out_specs=pl.BlockSpec((TN, 8), lambda i: (i, 0)),
        out_shape=jax.ShapeDtypeStruct((NP, 8), f32),
    )(s2, agg, scl, *ws)


def _pad_edges(ei):
    r = jnp.concatenate([ei[0], jnp.zeros((EP - E,), jnp.int32)])
    c = jnp.concatenate([ei[1], jnp.full((EP - E,), N, jnp.int32)])
    return r.reshape(R, 128), c.reshape(R, 128)


def kernel(x_game, x_state, edge_gg, edge_ss, edge_hist, edge_in,
           tag1_W, tag1_b, tag12_W, tag12_b, tag2_W, tag2_b, tag22_W, tag22_b,
           s3_Wl, s3_bl, s3_Wr, s32_Wl, s32_bl, s32_Wr,
           s4_Wl, s4_bl, s4_Wr, s42_Wl, s42_bl, s42_Wr, lin_W, lin_b):
    xg = jnp.zeros((NP, 16), f32).at[:N, :5].set(x_game)
    xs = jnp.zeros((NP, 16), f32).at[:N, :6].set(x_state)
    w1 = jnp.zeros((NHOPS + 1, 16, 64), f32).at[:, :5, :].set(tag1_W)
    w2 = jnp.zeros((NHOPS + 1, 16, 64), f32).at[:, :6, :].set(tag2_W)

    gg_r, gg_c = _pad_edges(edge_gg)
    ss_r, ss_c = _pad_edges(edge_ss)
    h_r, h_c = _pad_edges(edge_hist)
    i_r, i_c = _pad_edges(edge_in)

    ones1 = jnp.ones((128,), f32)
    zeros1 = jnp.zeros((SCH,), f32)
    zeros16 = jnp.zeros((SCH, 16), f32)

    cols4 = jnp.stack([jnp.stack([gg_c, h_c]), jnp.stack([ss_c, i_c])])
    deg4 = _degrees(cols4, ones1, zeros1).reshape(4, NP, 1)

    scl, drep2, q0n = _prep(deg4, xg, xs)

    rows_n = jnp.stack([gg_r, ss_r])
    cols_n = jnp.stack([gg_c, ss_c])
    qn = _propagate(1, rows_n, cols_n, q0n, drep2, zeros16)

    gx, gq0, sx, sq0 = _mmn(xg, xs, qn, scl, w1, tag1_b.reshape(1, 64),
                            w2, tag2_b.reshape(1, 64))

    drep_gg = jnp.stack([drep2[0]] * 4)
    drep_ss = jnp.stack([drep2[1]] * 4)
    gg_r4 = jnp.stack([gg_r] * 4)
    gg_c4 = jnp.stack([gg_c] * 4)
    ss_r4 = jnp.stack([ss_r] * 4)
    ss_c4 = jnp.stack([ss_c] * 4)

    qg = _propagate(2, gg_r4, gg_c4, gq0, drep_gg, zeros16)
    g2 = _mmw(0, gx, qg, scl, tag12_W, tag12_b.reshape(1, 64))

    qs = _propagate(2, ss_r4, ss_c4, sq0, drep_ss, zeros16)
    s2 = _mmw(1, sx, qs, scl, tag22_W, tag22_b.reshape(1, 64))

    h_r4 = jnp.stack([h_r] * 4)
    h_c4 = jnp.stack([h_c] * 4)
    i_r4 = jnp.stack([i_r] * 4)
    i_c4 = jnp.stack([i_c] * 4)
    agg = _sage_agg(h_r4, h_c4, i_r4, i_c4, g2, zeros16)

    out = _mms(s2, agg, scl,
               s3_Wl, s3_bl.reshape(1, 64), s3_Wr,
               s32_Wl, s32_bl.reshape(1, 64), s32_Wr,
               s4_Wl, s4_bl.reshape(1, 64), s4_Wr,
               s42_Wl, s42_bl.reshape(1, 64), s42_Wr,
               lin_W, lin_b.reshape(1, 8))
    return out[:N]


# speedup vs baseline: 1.2598x; 1.1671x over previous
import functools

import jax
import jax.numpy as jnp
from jax import lax
from jax.experimental import pallas as pl
from jax.experimental.pallas import tpu as pltpu
from jax.experimental.pallas import tpu_sc as plsc

N = 50000
NP = 50176
E = 800000
EP = 802816
R = EP // 128
RT = R // 16
PT = NP // 16
SCH = 56
SC2 = 448
EJ = 56
NHOPS = 10
TN = 256
NT = NP // TN

_mesh = plsc.VectorSubcoreMesh(core_axis_name="c", subcore_axis_name="s")
f32 = jnp.float32


def _degrees_body(cols4, ones_h, zeros_h, deg_out, cidx, cvec, ones, zbuf, acc):
    cid = lax.axis_index("c")
    sid = lax.axis_index("s")
    pltpu.sync_copy(ones_h, ones)
    pltpu.sync_copy(zeros_h, zbuf)
    for a in range(2):
        pltpu.sync_copy(cols4.at[cid, a, pl.ds(sid * RT, RT)], cidx)

        def _zero(m, carry):
            pltpu.sync_copy(zbuf, acc.at[pl.ds(sid * PT + m * SCH, SCH)])
            return carry

        lax.fori_loop(0, PT // SCH, _zero, 0)
        plsc.subcore_barrier()

        def _edges(j, carry):
            for t in range(8):
                cvec[pl.ds(t * 16, 16)] = cidx[j, pl.ds(t * 16, 16)]
            pltpu.sync_copy(ones, acc.at[cvec], add=True)
            return carry

        lax.fori_loop(0, RT, _edges, 0)
        plsc.subcore_barrier()
        pltpu.sync_copy(acc.at[pl.ds(sid * PT, PT)],
                        deg_out.at[cid, a, pl.ds(sid * PT, PT)])
        plsc.subcore_barrier()


def _degrees(cols4, ones_h, zeros_h):
    return pl.kernel(
        _degrees_body,
        out_type=jax.ShapeDtypeStruct((2, 2, NP), f32),
        mesh=_mesh,
        compiler_params=pltpu.CompilerParams(use_tc_tiling_on_sc=False),
        scratch_types=[
            pltpu.VMEM((RT, 128), jnp.int32),
            pltpu.VMEM((128,), jnp.int32),
            pltpu.VMEM((128,), f32),
            pltpu.VMEM((SCH,), f32),
            pltpu.VMEM_SHARED((NP,), f32),
        ],
    )(cols4, ones_h, zeros_h)


def _edge_pass(rows3, cols3, blk, src, acc, ridx, cidx, bufs, gsems, ssems,
               sid):
    def _echunk(c, c2):
        pltpu.sync_copy(rows3.at[blk, pl.ds(sid * RT + c * EJ, EJ)], ridx)
        pltpu.sync_copy(cols3.at[blk, pl.ds(sid * RT + c * EJ, EJ)], cidx)
        gd = [None] * 8
        sd = [None] * 8
        for j in range(8):
            gd[j] = pltpu.async_copy(src.at[ridx.at[j]], bufs[j], gsems[j])
        for j in range(EJ):
            b = j % 8
            gd[b].wait()
            sd[b] = pltpu.async_copy(bufs[b], acc.at[cidx.at[j]], ssems[b],
                                     add=True)
            if 4 <= j < EJ - 4:
                bn = (j + 4) % 8
                sd[bn].wait()
                gd[bn] = pltpu.async_copy(src.at[ridx.at[j + 4]], bufs[bn],
                                          gsems[bn])
        for b in range(8):
            sd[b].wait()
        return c2

    lax.fori_loop(0, RT // EJ, _echunk, 0)


def _propagate_body(NB, rows3, cols3, q0, drep, zeros_h, q_out,
                    ridx, cidx, b0, b1, b2, b3, b4, b5, b6, b7,
                    zbuf, acch, dreph, acc,
                    g0, g1, g2, g3, g4, g5, g6, g7,
                    s0, s1, s2, s3, s4, s5, s6, s7):
    cid = lax.axis_index("c")
    sid = lax.axis_index("s")
    bufs = [b0, b1, b2, b3, b4, b5, b6, b7]
    gsems = [g0, g1, g2, g3, g4, g5, g6, g7]
    ssems = [s0, s1, s2, s3, s4, s5, s6, s7]
    pltpu.sync_copy(zeros_h, zbuf)

    def _zero(m, c2):
        pltpu.sync_copy(zbuf, acc.at[pl.ds(sid * PT + m * SC2, SC2)])
        return c2

    lax.fori_loop(0, PT // SC2, _zero, 0)
    plsc.subcore_barrier()

    def _sub_pass(k, f):
        blk = cid * NB + f
        src = q0.at[blk] if k is None else q_out.at[k - 2, blk]

        _edge_pass(rows3, cols3, blk, src, acc, ridx, cidx, bufs,
                   gsems, ssems, sid)
        plsc.subcore_barrier()

        kk = 1 if k is None else k

        def _chunk(m, c2):
            r0 = sid * PT + m * SC2
            pltpu.sync_copy(acc.at[pl.ds(r0, SC2)], acch)
            pltpu.sync_copy(zbuf, acc.at[pl.ds(r0, SC2)])
            pltpu.sync_copy(drep.at[blk, pl.ds(r0, SC2)], dreph)

            def _scale(r, c3):
                acch[r] = acch[r] * dreph[r]
                return c3

            lax.fori_loop(0, SC2, _scale, 0)
            pltpu.sync_copy(acch, q_out.at[kk - 1, blk, pl.ds(r0, SC2)])
            return c2

        lax.fori_loop(0, PT // SC2, _chunk, 0)
        plsc.subcore_barrier()

    for f in range(NB):
        _sub_pass(None, f)

    def _hop(k, carry):
        for f in range(NB):
            _sub_pass(k, f)
        return carry

    lax.fori_loop(2, NHOPS + 1, _hop, 0)


def _propagate(NB, rows3, cols3, q0, drep, zeros_h):
    return pl.kernel(
        functools.partial(_propagate_body, NB),
        out_type=jax.ShapeDtypeStruct((NHOPS, 2 * NB, NP, 16), f32),
        mesh=_mesh,
        compiler_params=pltpu.CompilerParams(use_tc_tiling_on_sc=False),
        scratch_types=(
            [pltpu.VMEM((EJ, 128), jnp.int32)] * 2
            + [pltpu.VMEM((128, 16), f32)] * 8
            + [pltpu.VMEM((SC2, 16), f32)] * 3
            + [pltpu.VMEM_SHARED((NP, 16), f32)]
            + [pltpu.SemaphoreType.DMA] * 16
        ),
    )(rows3, cols3, q0, drep, zeros_h)


def _sage_body(rows_h, cols_h, rows_i, cols_i, gsrc, zeros_h, agg_out,
               ridx, cidx, b0, b1, b2, b3, b4, b5, b6, b7, zbuf, acc,
               g0, g1, g2, g3, g4, g5, g6, g7,
               s0, s1, s2, s3, s4, s5, s6, s7):
    cid = lax.axis_index("c")
    sid = lax.axis_index("s")
    bufs = [b0, b1, b2, b3, b4, b5, b6, b7]
    gsems = [g0, g1, g2, g3, g4, g5, g6, g7]
    ssems = [s0, s1, s2, s3, s4, s5, s6, s7]
    pltpu.sync_copy(zeros_h, zbuf)
    myslice = pl.ds(sid * PT, PT)

    def _zero(m, c2):
        pltpu.sync_copy(zbuf, acc.at[pl.ds(sid * PT + m * SC2, SC2)])
        return c2

    lax.fori_loop(0, PT // SC2, _zero, 0)
    plsc.subcore_barrier()
    for a in range(2):
        rows3 = rows_h if a == 0 else rows_i
        cols3 = cols_h if a == 0 else cols_i
        for f in range(2):
            blk = cid * 2 + f
            _edge_pass(rows3, cols3, blk, gsrc.at[blk], acc, ridx, cidx,
                       bufs, gsems, ssems, sid)
            plsc.subcore_barrier()
            pltpu.sync_copy(acc.at[myslice], agg_out.at[a, blk, myslice])
            lax.fori_loop(0, PT // SC2, _zero, 0)
            plsc.subcore_barrier()


def _sage_agg(rows_h, cols_h, rows_i, cols_i, g2, zeros_h):
    return pl.kernel(
        _sage_body,
        out_type=jax.ShapeDtypeStruct((2, 4, NP, 16), f32),
        mesh=_mesh,
        compiler_params=pltpu.CompilerParams(use_tc_tiling_on_sc=False),
        scratch_types=(
            [pltpu.VMEM((EJ, 128), jnp.int32)] * 2
            + [pltpu.VMEM((128, 16), f32)] * 8
            + [pltpu.VMEM((SC2, 16), f32)] * 1
            + [pltpu.VMEM_SHARED((NP, 16), f32)]
            + [pltpu.SemaphoreType.DMA] * 16
        ),
    )(rows_h, cols_h, rows_i, cols_i, g2, zeros_h)


def _prep_body(deg_ref, xg_ref, xs_ref, scl_ref, drep_ref, q0_ref):
    dgg = deg_ref[0]
    dh = deg_ref[1]
    dss = deg_ref[2]
    din = deg_ref[3]
    rs_gg = jnp.where(dgg > 0, lax.rsqrt(dgg), 0.0)
    rs_ss = jnp.where(dss > 0, lax.rsqrt(dss), 0.0)
    scl_ref[0] = jnp.where(dgg > 0, jnp.sqrt(dgg), 0.0)
    scl_ref[1] = jnp.where(dss > 0, jnp.sqrt(dss), 0.0)
    scl_ref[2] = 1.0 / jnp.maximum(dh, 1.0)
    scl_ref[3] = 1.0 / jnp.maximum(din, 1.0)
    scl_ref[4] = rs_gg
    scl_ref[5] = rs_ss
    drep_ref[0] = jnp.broadcast_to(jnp.where(dgg > 0, 1.0 / dgg, 0.0), (TN, 16))
    drep_ref[1] = jnp.broadcast_to(jnp.where(dss > 0, 1.0 / dss, 0.0), (TN, 16))
    q0_ref[0] = xg_ref[...] * rs_gg
    q0_ref[1] = xs_ref[...] * rs_ss


def _prep(deg4, xg, xs):
    return pl.pallas_call(
        _prep_body,
        grid=(NT,),
        in_specs=[
            pl.BlockSpec((4, TN, 1), lambda i: (0, i, 0)),
            pl.BlockSpec((TN, 16), lambda i: (i, 0)),
            pl.BlockSpec((TN, 16), lambda i: (i, 0)),
        ],
        out_specs=[
            pl.BlockSpec((6, TN, 1), lambda i: (0, i, 0)),
            pl.BlockSpec((2, TN, 16), lambda i: (0, i, 0)),
            pl.BlockSpec((2, TN, 16), lambda i: (0, i, 0)),
        ],
        out_shape=[
            jax.ShapeDtypeStruct((6, NP, 1), f32),
            jax.ShapeDtypeStruct((2, NP, 16), f32),
            jax.ShapeDtypeStruct((2, NP, 16), f32),
        ],
    )(deg4, xg, xs)


def _dot(a, b):
    return jnp.dot(a, b, preferred_element_type=f32)


def _mmn_body(xg_ref, xs_ref, qn_ref, scl_ref, w1_ref, b1_ref, w2_ref, b2_ref,
              gx_ref, gq0_ref, sx_ref, sq0_ref):
    for (x, w, b, d12, dinv, ox, oq, c) in (
            (xg_ref, w1_ref, b1_ref, scl_ref[0], scl_ref[4], gx_ref, gq0_ref, 0),
            (xs_ref, w2_ref, b2_ref, scl_ref[1], scl_ref[5], sx_ref, sq0_ref, 1)):
        acc0 = _dot(x[...], w[0])
        accp = _dot(qn_ref[0, c], w[1])
        for k in range(2, NHOPS + 1):
            accp = accp + _dot(qn_ref[k - 1, c], w[k])
        o = jax.nn.relu(acc0 + d12 * accp + b[...])
        oq2 = o * dinv
        for blk in range(4):
            ox[blk] = o[:, blk * 16:(blk + 1) * 16]
            oq[blk] = oq2[:, blk * 16:(blk + 1) * 16]


def _mmn(xg, xs, qn, scl, w1, b1, w2, b2):
    return pl.pallas_call(
        _mmn_body,
        grid=(NT,),
        in_specs=[
            pl.BlockSpec((TN, 16), lambda i: (i, 0)),
            pl.BlockSpec((TN, 16), lambda i: (i, 0)),
            pl.BlockSpec((NHOPS, 2, TN, 16), lambda i: (0, 0, i, 0)),
            pl.BlockSpec((6, TN, 1), lambda i: (0, i, 0)),
            pl.BlockSpec((NHOPS + 1, 16, 64), lambda i: (0, 0, 0)),
            pl.BlockSpec((1, 64), lambda i: (0, 0)),
            pl.BlockSpec((NHOPS + 1, 16, 64), lambda i: (0, 0, 0)),
            pl.BlockSpec((1, 64), lambda i: (0, 0)),
        ],
        out_specs=[pl.BlockSpec((4, TN, 16), lambda i: (0, i, 0))] * 4,
        out_shape=[jax.ShapeDtypeStruct((4, NP, 16), f32)] * 4,
    )(xg, xs, qn, scl, w1, b1, w2, b2)


def _mmw_body(srow, x_ref, q_ref, scl_ref, w_ref, b_ref, out_ref):
    acc0 = _dot(x_ref[0], w_ref[0, 0:16, :])
    for blk in range(1, 4):
        acc0 = acc0 + _dot(x_ref[blk], w_ref[0, blk * 16:(blk + 1) * 16, :])
    accp = None
    for k in range(1, NHOPS + 1):
        for blk in range(4):
            t = _dot(q_ref[k - 1, blk], w_ref[k, blk * 16:(blk + 1) * 16, :])
            accp = t if accp is None else accp + t
    o = jax.nn.relu(acc0 + scl_ref[srow] * accp + b_ref[...])
    for blk in range(4):
        out_ref[blk] = o[:, blk * 16:(blk + 1) * 16]


def _mmw(srow, x, q, scl, w, b):
    return pl.pallas_call(
        functools.partial(_mmw_body, srow),
        grid=(NT,),
        in_specs=[
            pl.BlockSpec((4, TN, 16), lambda i: (0, i, 0)),
            pl.BlockSpec((NHOPS, 4, TN, 16), lambda i: (0, 0, i, 0)),
            pl.BlockSpec((6, TN, 1), lambda i: (0, i, 0)),
            pl.BlockSpec((NHOPS + 1, 64, 64), lambda i: (0, 0, 0)),
            pl.BlockSpec((1, 64), lambda i: (0, 0)),
        ],
        out_specs=pl.BlockSpec((4, TN, 16), lambda i: (0, i, 0)),
        out_shape=jax.ShapeDtypeStruct((4, NP, 16), f32),
    )(x, q, scl, w, b)


def _mms_body(s2_ref, agg_ref, scl_ref,
              wl3, bl3, wr3, wl32, bl32, wr32,
              wl4, bl4, wr4, wl42, bl42, wr42, lw, lb, out_ref):
    s = jnp.concatenate([s2_ref[b] for b in range(4)], axis=1)
    mh = jnp.concatenate([agg_ref[0, b] for b in range(4)], axis=1) * scl_ref[2]
    mi = jnp.concatenate([agg_ref[1, b] for b in range(4)], axis=1) * scl_ref[3]
    s = jax.nn.relu(_dot(mh, wl3[...]) + bl3[...] + _dot(s, wr3[...]))
    s = jax.nn.relu(_dot(mh, wl32[...]) + bl32[...] + _dot(s, wr32[...]))
    s = jax.nn.relu(_dot(mi, wl4[...]) + bl4[...] + _dot(s, wr4[...]))
    s = jax.nn.relu(_dot(mi, wl42[...]) + bl42[...] + _dot(s, wr42[...]))
    out_ref[...] = _dot(s, lw[...]) + lb[...]


def _mms(s2, agg, scl, *ws):
    wspecs = []
    for w in ws:
        wspecs.append(pl.BlockSpec(w.shape, lambda i, nd=w.ndim: (0,) * nd))
    return pl.pallas_call(
        _mms_body,
        grid=(NT,),
        in_specs=[
            pl.BlockSpec((4, TN, 16), lambda i: (0, i, 0)),
            pl.BlockSpec((2, 4, TN, 16), lambda i: (0, 0, i, 0)),
            pl.BlockSpec((6, TN, 1), lambda i: (0, i, 0)),
        ] + wspecs,
        out_specs=pl.BlockSpec((TN, 8), lambda i: (i, 0)),
        out_shape=jax.ShapeDtypeStruct((NP, 8), f32),
    )(s2, agg, scl, *ws)


def _pad_edges(ei):
    r = jnp.concatenate([ei[0], jnp.zeros((EP - E,), jnp.int32)])
    c = jnp.concatenate([ei[1], jnp.full((EP - E,), N, jnp.int32)])
    return r.reshape(R, 128), c.reshape(R, 128)


def kernel(x_game, x_state, edge_gg, edge_ss, edge_hist, edge_in,
           tag1_W, tag1_b, tag12_W, tag12_b, tag2_W, tag2_b, tag22_W, tag22_b,
           s3_Wl, s3_bl, s3_Wr, s32_Wl, s32_bl, s32_Wr,
           s4_Wl, s4_bl, s4_Wr, s42_Wl, s42_bl, s42_Wr, lin_W, lin_b):
    xg = jnp.zeros((NP, 16), f32).at[:N, :5].set(x_game)
    xs = jnp.zeros((NP, 16), f32).at[:N, :6].set(x_state)
    w1 = jnp.zeros((NHOPS + 1, 16, 64), f32).at[:, :5, :].set(tag1_W)
    w2 = jnp.zeros((NHOPS + 1, 16, 64), f32).at[:, :6, :].set(tag2_W)

    gg_r, gg_c = _pad_edges(edge_gg)
    ss_r, ss_c = _pad_edges(edge_ss)
    h_r, h_c = _pad_edges(edge_hist)
    i_r, i_c = _pad_edges(edge_in)

    ones1 = jnp.ones((128,), f32)
    zeros1 = jnp.zeros((SCH,), f32)
    zeros16 = jnp.zeros((SC2, 16), f32)

    cols4 = jnp.stack([jnp.stack([gg_c, h_c]), jnp.stack([ss_c, i_c])])
    deg4 = _degrees(cols4, ones1, zeros1).reshape(4, NP, 1)

    scl, drep2, q0n = _prep(deg4, xg, xs)

    rows_n = jnp.stack([gg_r, ss_r])
    cols_n = jnp.stack([gg_c, ss_c])
    qn = _propagate(1, rows_n, cols_n, q0n, drep2, zeros16)

    gx, gq0, sx, sq0 = _mmn(xg, xs, qn, scl, w1, tag1_b.reshape(1, 64),
                            w2, tag2_b.reshape(1, 64))

    drep_gg = jnp.stack([drep2[0]] * 4)
    drep_ss = jnp.stack([drep2[1]] * 4)
    gg_r4 = jnp.stack([gg_r] * 4)
    gg_c4 = jnp.stack([gg_c] * 4)
    ss_r4 = jnp.stack([ss_r] * 4)
    ss_c4 = jnp.stack([ss_c] * 4)

    qg = _propagate(2, gg_r4, gg_c4, gq0, drep_gg, zeros16)
    g2 = _mmw(0, gx, qg, scl, tag12_W, tag12_b.reshape(1, 64))

    qs = _propagate(2, ss_r4, ss_c4, sq0, drep_ss, zeros16)
    s2 = _mmw(1, sx, qs, scl, tag22_W, tag22_b.reshape(1, 64))

    h_r4 = jnp.stack([h_r] * 4)
    h_c4 = jnp.stack([h_c] * 4)
    i_r4 = jnp.stack([i_r] * 4)
    i_c4 = jnp.stack([i_c] * 4)
    agg = _sage_agg(h_r4, h_c4, i_r4, i_c4, g2, zeros16)

    out = _mms(s2, agg, scl,
               s3_Wl, s3_bl.reshape(1, 64), s3_Wr,
               s32_Wl, s32_bl.reshape(1, 64), s32_Wr,
               s4_Wl, s4_bl.reshape(1, 64), s4_Wr,
               s42_Wl, s42_bl.reshape(1, 64), s42_Wr,
               lin_W, lin_b.reshape(1, 8))
    return out[:N]
